# baseline (speedup 1.0000x reference)
import functools

import jax
import jax.numpy as jnp
from jax import lax
from jax.experimental import pallas as pl
from jax.experimental.pallas import tpu as pltpu
from jax.experimental.pallas import tpu_sc as plsc

N = 10000
E = 320000
D = 128
G = 256

NC = 2
NS = 16
NW = NC * NS

NPAD = 10240
RPT = NPAD // NS
EPW = E // NW
K = 80
NCHUNK = EPW // K

BN = 2048
NB = NPAD // BN


_sc_mesh = plsc.VectorSubcoreMesh(
    core_axis_name="c", subcore_axis_name="s", num_cores=NC, num_subcores=NS
)


@functools.partial(
    pl.kernel,
    out_type=jax.ShapeDtypeStruct((NC * NPAD, D), jnp.float32),
    mesh=_sc_mesh,
    scratch_types=[
        pltpu.VMEM_SHARED((NPAD, D), jnp.float32),
        pltpu.VMEM((EPW,), jnp.int32),
        pltpu.VMEM((NCHUNK, K), jnp.int32),
        pltpu.VMEM((K, D), jnp.float32),
        pltpu.VMEM((K, D), jnp.float32),
        pltpu.SemaphoreType.DMA,
        pltpu.SemaphoreType.DMA,
    ],
)
def _sc_agg(h_hbm, src_hbm, dst_hbm, zero_hbm, out_hbm,
            acc, srcs_v, dsts_v, rows0, rows1, gsem0, gsem1):
    c = lax.axis_index("c")
    s = lax.axis_index("s")
    wid = s * NC + c
    row0 = s * RPT

    pltpu.sync_copy(src_hbm.at[wid], srcs_v)
    pltpu.sync_copy(dst_hbm.at[wid], dsts_v)
    pltpu.sync_copy(zero_hbm, rows0)
    for j in range(RPT // K):
        pltpu.sync_copy(rows0, acc.at[pl.ds(row0 + j * K, K)])
    plsc.subcore_barrier()

    def _sidx(cc):
        return srcs_v.at[pl.ds(cc * K, K)]

    K2 = K // 2

    def _fire(cc, buf, sem):
        pltpu.async_copy(
            h_hbm.at[srcs_v.at[pl.ds(cc * K, K2)]], buf.at[pl.ds(0, K2)], sem
        )
        pltpu.async_copy(
            h_hbm.at[srcs_v.at[pl.ds(cc * K + K2, K2)]],
            buf.at[pl.ds(K2, K2)],
            sem,
        )

    _fire(0, rows0, gsem0)

    def _half(cc, cur, nxt, cur_sem, nxt_sem):
        _fire(cc + 1, nxt, nxt_sem)
        pltpu.make_async_copy(h_hbm.at[_sidx(cc)], cur, cur_sem).wait()
        pltpu.sync_copy(cur, acc.at[dsts_v.at[cc]], add=True)

    def _pair(j, carry):
        c0 = 2 * j
        _half(c0, rows0, rows1, gsem0, gsem1)
        _half(c0 + 1, rows1, rows0, gsem1, gsem0)
        return carry

    lax.fori_loop(0, (NCHUNK - 1) // 2, _pair, 0)
    last = NCHUNK - 1
    pltpu.make_async_copy(h_hbm.at[_sidx(last)], rows0, gsem0).wait()
    pltpu.sync_copy(rows0, acc.at[dsts_v.at[last]], add=True)

    plsc.subcore_barrier()
    out0 = c * NPAD + row0
    for j in range(RPT // K):
        pltpu.sync_copy(acc.at[pl.ds(row0 + j * K, K)], rows0)
        pltpu.sync_copy(rows0, out_hbm.at[pl.ds(out0 + j * K, K)])


def _embed_body(x_ref, w_ref, b_ref, o_ref):
    o_ref[...] = (
        jnp.dot(x_ref[...], w_ref[...], preferred_element_type=jnp.float32)
        + b_ref[...]
    )


_embed = pl.pallas_call(
    _embed_body,
    grid=(NB,),
    in_specs=[
        pl.BlockSpec((BN, D), lambda i: (i, 0)),
        pl.BlockSpec((D, D), lambda i: (0, 0)),
        pl.BlockSpec((1, D), lambda i: (0, 0)),
    ],
    out_specs=pl.BlockSpec((BN, D), lambda i: (i, 0)),
    out_shape=jax.ShapeDtypeStruct((NPAD, D), jnp.float32),
)


def _mlp_body(h_ref, a0_ref, a1_ref, w1_ref, b1_ref, w2_ref, b2_ref, o_ref,
              *, relu_out):
    m = h_ref[...] + a0_ref[...] + a1_ref[...]
    t = jnp.maximum(
        jnp.dot(m, w1_ref[...], preferred_element_type=jnp.float32)
        + b1_ref[...],
        0.0,
    )
    r = (
        jnp.dot(t, w2_ref[...], preferred_element_type=jnp.float32)
        + b2_ref[...]
    )
    if relu_out:
        r = jnp.maximum(r, 0.0)
    o_ref[...] = r


def _make_mlp(relu_out):
    return pl.pallas_call(
        functools.partial(_mlp_body, relu_out=relu_out),
        grid=(NB,),
        in_specs=[
            pl.BlockSpec((BN, D), lambda i: (i, 0)),
            pl.BlockSpec((BN, D), lambda i: (i, 0)),
            pl.BlockSpec((BN, D), lambda i: (i + NB, 0)),
            pl.BlockSpec((D, D), lambda i: (0, 0)),
            pl.BlockSpec((1, D), lambda i: (0, 0)),
            pl.BlockSpec((D, D), lambda i: (0, 0)),
            pl.BlockSpec((1, D), lambda i: (0, 0)),
        ],
        out_specs=pl.BlockSpec((BN, D), lambda i: (i, 0)),
        out_shape=jax.ShapeDtypeStruct((NPAD, D), jnp.float32),
    )


_mlp_relu = _make_mlp(True)


def _mlp_pool_body(h_ref, a0_ref, a1_ref, w1_ref, b1_ref, w2_ref, b2_ref,
                   b_ref, wt1_ref, bt1_ref, wt2_ref, bt2_ref, o_ref,
                   sums_ref, cnts_ref):
    i = pl.program_id(0)

    @pl.when(i == 0)
    def _init():
        sums_ref[...] = jnp.zeros_like(sums_ref)
        cnts_ref[...] = jnp.zeros_like(cnts_ref)

    m = h_ref[...] + a0_ref[...] + a1_ref[...]
    t = jnp.maximum(
        jnp.dot(m, w1_ref[...], preferred_element_type=jnp.float32)
        + b1_ref[...],
        0.0,
    )
    r = (
        jnp.dot(t, w2_ref[...], preferred_element_type=jnp.float32)
        + b2_ref[...]
    )
    b = b_ref[0, :, :]
    oh = (b == lax.broadcasted_iota(jnp.int32, (G, BN), 0)).astype(jnp.float32)
    sums_ref[...] += jnp.dot(oh, r, preferred_element_type=jnp.float32)
    cnts_ref[...] += jnp.sum(oh, axis=1, keepdims=True)

    @pl.when(i == NB - 1)
    def _fin():
        pooled = sums_ref[...] / jnp.maximum(cnts_ref[...], 1.0)
        tt = jnp.maximum(
            jnp.dot(pooled, wt1_ref[...], preferred_element_type=jnp.float32)
            + bt1_ref[...],
            0.0,
        )
        o_ref[...] = (
            jnp.dot(tt, wt2_ref[...], preferred_element_type=jnp.float32)
            + bt2_ref[...]
        )


_mlp_pool = pl.pallas_call(
    _mlp_pool_body,
    grid=(NB,),
    in_specs=[
        pl.BlockSpec((BN, D), lambda i: (i, 0)),
        pl.BlockSpec((BN, D), lambda i: (i, 0)),
        pl.BlockSpec((BN, D), lambda i: (i + NB, 0)),
        pl.BlockSpec((D, D), lambda i: (0, 0)),
        pl.BlockSpec((1, D), lambda i: (0, 0)),
        pl.BlockSpec((D, D), lambda i: (0, 0)),
        pl.BlockSpec((1, D), lambda i: (0, 0)),
        pl.BlockSpec((1, 1, BN), lambda i: (i, 0, 0)),
        pl.BlockSpec((D, D), lambda i: (0, 0)),
        pl.BlockSpec((1, D), lambda i: (0, 0)),
        pl.BlockSpec((D, 1), lambda i: (0, 0)),
        pl.BlockSpec((1, 1), lambda i: (0, 0)),
    ],
    out_specs=pl.BlockSpec((G, 1), lambda i: (0, 0)),
    out_shape=jax.ShapeDtypeStruct((G, 1), jnp.float32),
    scratch_shapes=[
        pltpu.VMEM((G, D), jnp.float32),
        pltpu.VMEM((G, D), jnp.float32),
    ],
)


def kernel(x, edge_index, batch, We, be,
           W1_0, b1_0, W2_0, b2_0,
           W1_1, b1_1, W2_1, b2_1,
           W1_2, b1_2, W2_2, b2_2,
           Wt1, bt1, Wt2, bt2):
    src = edge_index[0].reshape(NW, EPW)
    dst = edge_index[1].reshape(NW, NCHUNK, K)
    xp = jnp.zeros((NPAD, D), jnp.float32).at[:N].set(x)
    batchp = (
        jnp.full((NPAD,), G, jnp.int32).at[:N].set(batch).reshape(NB, 1, BN)
    )
    zrows = jnp.zeros((K, D), jnp.float32)

    h = _embed(xp, We, be.reshape(1, D))
    for W1, b1, W2, b2 in ((W1_0, b1_0, W2_0, b2_0), (W1_1, b1_1, W2_1, b2_1)):
        agg = _sc_agg(h, src, dst, zrows)
        h = _mlp_relu(h, agg, agg, W1, b1.reshape(1, D), W2, b2.reshape(1, D))

    agg = _sc_agg(h, src, dst, zrows)
    return _mlp_pool(
        h, agg, agg, W1_2, b1_2.reshape(1, D), W2_2, b2_2.reshape(1, D),
        batchp, Wt1, bt1.reshape(1, D), Wt2, bt2.reshape(1, 1),
    )


# speedup vs baseline: 1.0420x; 1.0420x over previous
import functools

import jax
import jax.numpy as jnp
from jax import lax
from jax.experimental import pallas as pl
from jax.experimental.pallas import tpu as pltpu
from jax.experimental.pallas import tpu_sc as plsc

N = 10000
E = 320000
D = 128
G = 256

NC = 2
NS = 16
NW = NC * NS

NPAD = 10240
RPT = NPAD // NS
EPW = E // NW
K = 80
NCHUNK = EPW // K

BN = 2048
NB = NPAD // BN


_sc_mesh = plsc.VectorSubcoreMesh(
    core_axis_name="c", subcore_axis_name="s", num_cores=NC, num_subcores=NS
)


@functools.partial(
    pl.kernel,
    out_type=jax.ShapeDtypeStruct((NC * NPAD, D), jnp.float32),
    mesh=_sc_mesh,
    scratch_types=[
        pltpu.VMEM_SHARED((NPAD, D), jnp.float32),
        pltpu.VMEM((EPW,), jnp.int32),
        pltpu.VMEM((NCHUNK, K), jnp.int32),
        pltpu.VMEM((K, D), jnp.float32),
        pltpu.VMEM((K, D), jnp.float32),
        pltpu.SemaphoreType.DMA,
        pltpu.SemaphoreType.DMA,
        pltpu.SemaphoreType.DMA,
        pltpu.SemaphoreType.DMA,
    ],
)
def _sc_agg(h_hbm, src_hbm, dst_hbm, zero_hbm, out_hbm,
            acc, srcs_v, dsts_v, rows0, rows1, gsem0, gsem1, wsem0, wsem1):
    c = lax.axis_index("c")
    s = lax.axis_index("s")
    wid = s * NC + c
    row0 = s * RPT
    rows = (rows0, rows1)
    gsems = (gsem0, gsem1)
    wsems = (wsem0, wsem1)
    nwrit = RPT // K

    pltpu.sync_copy(src_hbm.at[wid], srcs_v)
    pltpu.sync_copy(dst_hbm.at[wid], dsts_v)
    pltpu.sync_copy(zero_hbm, rows1)

    def _sidx(cc):
        return srcs_v.at[pl.ds(cc * K, K)]

    pltpu.async_copy(h_hbm.at[_sidx(0)], rows0, gsem0)
    for j in range(nwrit):
        pltpu.async_copy(rows1, acc.at[pl.ds(row0 + j * K, K)], wsem0)
    for j in range(nwrit):
        pltpu.make_async_copy(
            rows1, acc.at[pl.ds(row0 + j * K, K)], wsem0
        ).wait()
    plsc.subcore_barrier()

    def _half(cc, cur, nxt, cur_sem, nxt_sem):
        pltpu.async_copy(h_hbm.at[_sidx(cc + 1)], nxt, nxt_sem)
        pltpu.make_async_copy(h_hbm.at[_sidx(cc)], cur, cur_sem).wait()
        pltpu.sync_copy(cur, acc.at[dsts_v.at[cc]], add=True)

    def _pair(j, carry):
        c0 = 2 * j
        _half(c0, rows0, rows1, gsem0, gsem1)
        _half(c0 + 1, rows1, rows0, gsem1, gsem0)
        return carry

    lax.fori_loop(0, (NCHUNK - 1) // 2, _pair, 0)
    last = NCHUNK - 1
    pltpu.make_async_copy(h_hbm.at[_sidx(last)], rows0, gsem0).wait()
    pltpu.sync_copy(rows0, acc.at[dsts_v.at[last]], add=True)

    plsc.subcore_barrier()

    out0 = c * NPAD + row0

    def _acc_rows(j):
        return acc.at[pl.ds(row0 + j * K, K)]

    def _out_rows(j):
        return out_hbm.at[pl.ds(out0 + j * K, K)]

    pltpu.async_copy(_acc_rows(0), rows0, gsem0)
    for j in range(nwrit):
        b = j % 2
        pltpu.make_async_copy(_acc_rows(j), rows[b], gsems[b]).wait()
        pltpu.async_copy(rows[b], _out_rows(j), wsems[b])
        if j + 1 < nwrit:
            if j >= 1:
                pltpu.make_async_copy(
                    rows[1 - b], _out_rows(j - 1), wsems[1 - b]
                ).wait()
            pltpu.async_copy(_acc_rows(j + 1), rows[1 - b], gsems[1 - b])
    pltpu.make_async_copy(
        rows[(nwrit - 1) % 2], _out_rows(nwrit - 1), wsems[(nwrit - 1) % 2]
    ).wait()


def _embed_body(x_ref, w_ref, b_ref, o_ref):
    o_ref[...] = (
        jnp.dot(x_ref[...], w_ref[...], preferred_element_type=jnp.float32)
        + b_ref[...]
    )


_embed = pl.pallas_call(
    _embed_body,
    grid=(NB,),
    in_specs=[
        pl.BlockSpec((BN, D), lambda i: (i, 0)),
        pl.BlockSpec((D, D), lambda i: (0, 0)),
        pl.BlockSpec((1, D), lambda i: (0, 0)),
    ],
    out_specs=pl.BlockSpec((BN, D), lambda i: (i, 0)),
    out_shape=jax.ShapeDtypeStruct((NPAD, D), jnp.float32),
)


def _mlp_body(h_ref, a0_ref, a1_ref, w1_ref, b1_ref, w2_ref, b2_ref, o_ref,
              *, relu_out):
    m = h_ref[...] + a0_ref[...] + a1_ref[...]
    t = jnp.maximum(
        jnp.dot(m, w1_ref[...], preferred_element_type=jnp.float32)
        + b1_ref[...],
        0.0,
    )
    r = (
        jnp.dot(t, w2_ref[...], preferred_element_type=jnp.float32)
        + b2_ref[...]
    )
    if relu_out:
        r = jnp.maximum(r, 0.0)
    o_ref[...] = r


def _make_mlp(relu_out):
    return pl.pallas_call(
        functools.partial(_mlp_body, relu_out=relu_out),
        grid=(NB,),
        in_specs=[
            pl.BlockSpec((BN, D), lambda i: (i, 0)),
            pl.BlockSpec((BN, D), lambda i: (i, 0)),
            pl.BlockSpec((BN, D), lambda i: (i + NB, 0)),
            pl.BlockSpec((D, D), lambda i: (0, 0)),
            pl.BlockSpec((1, D), lambda i: (0, 0)),
            pl.BlockSpec((D, D), lambda i: (0, 0)),
            pl.BlockSpec((1, D), lambda i: (0, 0)),
        ],
        out_specs=pl.BlockSpec((BN, D), lambda i: (i, 0)),
        out_shape=jax.ShapeDtypeStruct((NPAD, D), jnp.float32),
    )


_mlp_relu = _make_mlp(True)


def _mlp_pool_body(h_ref, a0_ref, a1_ref, w1_ref, b1_ref, w2_ref, b2_ref,
                   b_ref, wt1_ref, bt1_ref, wt2_ref, bt2_ref, o_ref,
                   sums_ref, cnts_ref):
    i = pl.program_id(0)

    @pl.when(i == 0)
    def _init():
        sums_ref[...] = jnp.zeros_like(sums_ref)
        cnts_ref[...] = jnp.zeros_like(cnts_ref)

    m = h_ref[...] + a0_ref[...] + a1_ref[...]
    t = jnp.maximum(
        jnp.dot(m, w1_ref[...], preferred_element_type=jnp.float32)
        + b1_ref[...],
        0.0,
    )
    r = (
        jnp.dot(t, w2_ref[...], preferred_element_type=jnp.float32)
        + b2_ref[...]
    )
    b = b_ref[0, :, :]
    oh = (b == lax.broadcasted_iota(jnp.int32, (G, BN), 0)).astype(jnp.float32)
    sums_ref[...] += jnp.dot(oh, r, preferred_element_type=jnp.float32)
    cnts_ref[...] += jnp.sum(oh, axis=1, keepdims=True)

    @pl.when(i == NB - 1)
    def _fin():
        pooled = sums_ref[...] / jnp.maximum(cnts_ref[...], 1.0)
        tt = jnp.maximum(
            jnp.dot(pooled, wt1_ref[...], preferred_element_type=jnp.float32)
            + bt1_ref[...],
            0.0,
        )
        o_ref[...] = (
            jnp.dot(tt, wt2_ref[...], preferred_element_type=jnp.float32)
            + bt2_ref[...]
        )


_mlp_pool = pl.pallas_call(
    _mlp_pool_body,
    grid=(NB,),
    in_specs=[
        pl.BlockSpec((BN, D), lambda i: (i, 0)),
        pl.BlockSpec((BN, D), lambda i: (i, 0)),
        pl.BlockSpec((BN, D), lambda i: (i + NB, 0)),
        pl.BlockSpec((D, D), lambda i: (0, 0)),
        pl.BlockSpec((1, D), lambda i: (0, 0)),
        pl.BlockSpec((D, D), lambda i: (0, 0)),
        pl.BlockSpec((1, D), lambda i: (0, 0)),
        pl.BlockSpec((1, 1, BN), lambda i: (i, 0, 0)),
        pl.BlockSpec((D, D), lambda i: (0, 0)),
        pl.BlockSpec((1, D), lambda i: (0, 0)),
        pl.BlockSpec((D, 1), lambda i: (0, 0)),
        pl.BlockSpec((1, 1), lambda i: (0, 0)),
    ],
    out_specs=pl.BlockSpec((G, 1), lambda i: (0, 0)),
    out_shape=jax.ShapeDtypeStruct((G, 1), jnp.float32),
    scratch_shapes=[
        pltpu.VMEM((G, D), jnp.float32),
        pltpu.VMEM((G, D), jnp.float32),
    ],
)


def kernel(x, edge_index, batch, We, be,
           W1_0, b1_0, W2_0, b2_0,
           W1_1, b1_1, W2_1, b2_1,
           W1_2, b1_2, W2_2, b2_2,
           Wt1, bt1, Wt2, bt2):
    src = edge_index[0].reshape(NW, EPW)
    dst = edge_index[1].reshape(NW, NCHUNK, K)
    xp = jnp.zeros((NPAD, D), jnp.float32).at[:N].set(x)
    batchp = (
        jnp.full((NPAD,), G, jnp.int32).at[:N].set(batch).reshape(NB, 1, BN)
    )
    zrows = jnp.zeros((K, D), jnp.float32)

    h = _embed(xp, We, be.reshape(1, D))
    for W1, b1, W2, b2 in ((W1_0, b1_0, W2_0, b2_0), (W1_1, b1_1, W2_1, b2_1)):
        agg = _sc_agg(h, src, dst, zrows)
        h = _mlp_relu(h, agg, agg, W1, b1.reshape(1, D), W2, b2.reshape(1, D))

    agg = _sc_agg(h, src, dst, zrows)
    return _mlp_pool(
        h, agg, agg, W1_2, b1_2.reshape(1, D), W2_2, b2_2.reshape(1, D),
        batchp, Wt1, bt1.reshape(1, D), Wt2, bt2.reshape(1, 1),
    )
